# baseline (speedup 1.0000x reference)
import jax
import jax.numpy as jnp
from jax import lax
from jax.experimental import pallas as pl
from jax.experimental.pallas import tpu as pltpu
from jax.experimental.pallas import tpu_sc as plsc

_B, _T, _S = 16, 2048, 8
_L = 16
_TC = 128
_NC = _T // _TC
_NORM = 8


def _gather16(v, idx):
    dnums = lax.GatherDimensionNumbers(
        offset_dims=(), collapsed_slice_dims=(0,), start_index_map=(0,))
    return lax.gather(v, idx[:, None], dnums, slice_sizes=(1,),
                      mode=lax.GatherScatterMode.PROMISE_IN_BOUNDS)


def _sc_recur(logB_hbm, pf_hbm, pb_hbm, alpha_hbm, beta_hbm, y_hbm,
              locB, res, resy, ptab):
    b = lax.axis_index("s")
    dire = lax.axis_index("c")
    locB[pl.ds(_T * _S, _L)] = jnp.zeros((_L,), jnp.float32)
    pltpu.sync_copy(logB_hbm.at[b], locB.at[pl.ds(0, _T * _S)])

    lanes = lax.iota(jnp.int32, _L)
    mask8 = lanes < _S
    pair_idx = [jnp.where(mask8, p, p + 4) for p in range(_S // 2)]
    hi_idx = jnp.where(mask8, lanes + _S, lanes)
    zeros_i = jnp.zeros((_L,), jnp.int32)

    def matvec(v, tab):
        acc = _gather16(v, pair_idx[0]) * tab[0]
        for p in range(1, _S // 2):
            acc = acc + _gather16(v, pair_idx[p]) * tab[p]
        return acc + _gather16(acc, hi_idx)

    @pl.when(dire == 0)
    def _fwd():
        pltpu.sync_copy(pf_hbm, ptab)
        tab = tuple(ptab[p] for p in range(_S // 2))
        a0 = jnp.exp(locB[pl.ds(0, _L)])
        res[pl.ds(0, _L)] = a0

        def step(t, u):
            vB = jnp.exp(locB[pl.ds(8 * t, _L)])
            al = u * vB
            res[pl.ds(16 * t, _L)] = al
            return matvec(al, tab)

        def blk(i, u):
            tt = 2 * _NORM * i + 1
            for v in range(2):
                for w in range(_NORM):
                    u = step(tt + v * _NORM + w, u)
                u = u / _gather16(u, zeros_i)
            return u

        u = lax.fori_loop(0, (_T - 1) // (2 * _NORM), blk, matvec(a0, tab))
        for k, t in enumerate(range(_T - 1 - (_T - 1) % (2 * _NORM) + 1, _T)):
            u = step(t, u)
            if k == _NORM - 1:
                u = u / _gather16(u, zeros_i)
        pltpu.sync_copy(res, alpha_hbm.at[b])

    @pl.when(dire == 1)
    def _bwd():
        pltpu.sync_copy(pb_hbm, ptab)
        tab = tuple(ptab[p] for p in range(_S // 2))
        bv = jnp.where(mask8, 1.0, 0.0)
        res[pl.ds(16 * (_T - 1), _L)] = bv

        def step(t, bv):
            vB = jnp.exp(locB[pl.ds(8 * (t + 1), _L)])
            y = bv * vB
            resy[pl.ds(16 * t, _L)] = y
            b2 = matvec(y, tab)
            res[pl.ds(16 * t, _L)] = b2
            return b2

        def blk(i, bv):
            tt = _T - 2 - 2 * _NORM * i
            for v in range(2):
                for w in range(_NORM):
                    bv = step(tt - v * _NORM - w, bv)
                bv = bv / _gather16(bv, zeros_i)
            return bv

        bv = lax.fori_loop(0, (_T - 1) // (2 * _NORM), blk, bv)
        for k, t in enumerate(range((_T - 1) % (2 * _NORM) - 1, -1, -1)):
            bv = step(t, bv)
            if k == _NORM - 1:
                bv = bv / _gather16(bv, zeros_i)
        pltpu.sync_copy(res, beta_hbm.at[b])
        pltpu.sync_copy(resy, y_hbm.at[b])


def _run_sc_recur(log_B, trans_prob):
    P = jax.lax.stop_gradient(trans_prob).astype(jnp.float32)
    pf = jnp.concatenate([P[: _S // 2], P[_S // 2:]], axis=1)
    pb = jnp.concatenate([P.T[: _S // 2], P.T[_S // 2:]], axis=1)
    logB_flat = log_B.reshape(_B, _T * _S)

    mesh = plsc.VectorSubcoreMesh(core_axis_name="c", subcore_axis_name="s")
    alpha, beta, ylin = pl.kernel(
        _sc_recur,
        out_type=[jax.ShapeDtypeStruct((_B, _T * _L), jnp.float32)] * 3,
        mesh=mesh,
        scratch_types=[
            pltpu.VMEM((_T * _S + _L,), jnp.float32),
            pltpu.VMEM((_T * _L,), jnp.float32),
            pltpu.VMEM((_T * _L,), jnp.float32),
            pltpu.VMEM((_S // 2, _L), jnp.float32),
        ],
    )(logB_flat, pf, pb)
    return alpha, beta, ylin


def _deint(x2):
    return jnp.swapaxes(x2.reshape(_B, _TC, _L), 1, 2)[:, :_S, :]


def _finalize_kernel(alpha_ref, beta_ref, ylin_ref, lp_ref, gamma_ref, xi_ref):
    la = jnp.log(_deint(alpha_ref[...]))
    lb = jnp.log(_deint(beta_ref[...]))

    g = la + lb
    g = g - jnp.max(g, axis=1, keepdims=True)
    g = g - jnp.log(jnp.sum(jnp.exp(g), axis=1, keepdims=True))
    gamma_ref[...] = jnp.swapaxes(g, 1, 2)

    ly = jnp.log(_deint(ylin_ref[...]))
    z = (la[:, :, None, :] + ly[:, None, :, :]
         + lp_ref[...][None, :, :, None]).reshape(_B, _S * _S, _TC)
    z = z - jnp.max(z, axis=1, keepdims=True)
    z = z - jnp.log(jnp.sum(jnp.exp(z), axis=1, keepdims=True))
    xi_ref[...] = jnp.swapaxes(z, 1, 2)


def _run_finalize(alpha, beta, ylin, log_P):
    cs = pl.BlockSpec((_B, _TC * _L), lambda c: (0, c))
    gamma, xi64 = pl.pallas_call(
        _finalize_kernel,
        grid=(_NC,),
        in_specs=[cs, cs, cs, pl.BlockSpec((_S, _S), lambda c: (0, 0))],
        out_specs=[pl.BlockSpec((_B, _TC, _S), lambda c: (0, c, 0)),
                   pl.BlockSpec((_B, _TC, _S * _S), lambda c: (0, c, 0))],
        out_shape=[
            jax.ShapeDtypeStruct((_B, _T, _S), jnp.float32),
            jax.ShapeDtypeStruct((_B, _T - 1, _S * _S), jnp.float32),
        ],
    )(alpha, beta, ylin, log_P)
    return gamma, xi64.reshape(_B, _T - 1, _S, _S)


def kernel(log_B, trans_prob):
    log_B = log_B.astype(jnp.float32)
    log_P = jnp.log(jax.lax.stop_gradient(trans_prob)).astype(jnp.float32)
    alpha, beta, ylin = _run_sc_recur(log_B, trans_prob)
    return _run_finalize(alpha, beta, ylin, log_P)


# speedup vs baseline: 1.0079x; 1.0079x over previous
import jax
import jax.numpy as jnp
from jax import lax
from jax.experimental import pallas as pl
from jax.experimental.pallas import tpu as pltpu
from jax.experimental.pallas import tpu_sc as plsc

_B, _T, _S = 16, 2048, 8
_L = 16
_TC = 128
_NC = _T // _TC
_NORM = 8


def _gather16(v, idx):
    dnums = lax.GatherDimensionNumbers(
        offset_dims=(), collapsed_slice_dims=(0,), start_index_map=(0,))
    return lax.gather(v, idx[:, None], dnums, slice_sizes=(1,),
                      mode=lax.GatherScatterMode.PROMISE_IN_BOUNDS)


def _sc_recur(logB_hbm, pf_hbm, pb_hbm, alpha_hbm, beta_hbm, y_hbm,
              locB, res, resy, ptab):
    b = lax.axis_index("s")
    dire = lax.axis_index("c")
    locB[pl.ds(_T * _S, _L)] = jnp.zeros((_L,), jnp.float32)
    pltpu.sync_copy(logB_hbm.at[b], locB.at[pl.ds(0, _T * _S)])

    lanes = lax.iota(jnp.int32, _L)
    mask8 = lanes < _S
    pair_idx = [jnp.where(mask8, p, p + 4) for p in range(_S // 2)]
    hi_idx = jnp.where(mask8, lanes + _S, lanes)
    zeros_i = jnp.zeros((_L,), jnp.int32)

    def matvec(v, tab):
        acc = _gather16(v, pair_idx[0]) * tab[0]
        for p in range(1, _S // 2):
            acc = acc + _gather16(v, pair_idx[p]) * tab[p]
        return acc + _gather16(acc, hi_idx)

    @pl.when(dire == 0)
    def _fwd():
        pltpu.sync_copy(pf_hbm, ptab)
        tab = tuple(ptab[p] for p in range(_S // 2))
        a = jnp.exp(locB[pl.ds(0, _L)])
        res[pl.ds(0, _L)] = a

        def step(t, a):
            vB = jnp.exp(locB[pl.ds(8 * t, _L)])
            a2 = matvec(a, tab) * vB
            res[pl.ds(16 * t, _L)] = a2
            return a2

        def blk(i, a):
            tt = _NORM * i + 1
            for u in range(_NORM):
                a = step(tt + u, a)
            return a / _gather16(a, zeros_i)

        a = lax.fori_loop(0, (_T - 1) // _NORM, blk, a)
        for t in range(_T - 1 - (_T - 1) % _NORM + 1, _T):
            a = step(t, a)
        pltpu.sync_copy(res, alpha_hbm.at[b])

    @pl.when(dire == 1)
    def _bwd():
        pltpu.sync_copy(pb_hbm, ptab)
        tab = tuple(ptab[p] for p in range(_S // 2))
        bv = jnp.where(mask8, 1.0, 0.0)
        res[pl.ds(16 * (_T - 1), _L)] = bv

        def step(t, bv):
            vB = jnp.exp(locB[pl.ds(8 * (t + 1), _L)])
            y = bv * vB
            resy[pl.ds(16 * t, _L)] = y
            b2 = matvec(y, tab)
            res[pl.ds(16 * t, _L)] = b2
            return b2

        def blk(i, bv):
            tt = _T - 2 - _NORM * i
            for u in range(_NORM):
                bv = step(tt - u, bv)
            return bv / _gather16(bv, zeros_i)

        bv = lax.fori_loop(0, (_T - 1) // _NORM, blk, bv)
        for t in range((_T - 1) % _NORM - 1, -1, -1):
            bv = step(t, bv)
        pltpu.sync_copy(res, beta_hbm.at[b])
        pltpu.sync_copy(resy, y_hbm.at[b])


def _run_sc_recur(log_B, trans_prob):
    P = jax.lax.stop_gradient(trans_prob).astype(jnp.float32)
    pf = jnp.concatenate([P[: _S // 2], P[_S // 2:]], axis=1)
    pb = jnp.concatenate([P.T[: _S // 2], P.T[_S // 2:]], axis=1)
    logB_flat = log_B.reshape(_B, _T * _S)

    mesh = plsc.VectorSubcoreMesh(core_axis_name="c", subcore_axis_name="s")
    alpha, beta, ylin = pl.kernel(
        _sc_recur,
        out_type=[jax.ShapeDtypeStruct((_B, _T * _L), jnp.float32)] * 3,
        mesh=mesh,
        scratch_types=[
            pltpu.VMEM((_T * _S + _L,), jnp.float32),
            pltpu.VMEM((_T * _L,), jnp.float32),
            pltpu.VMEM((_T * _L,), jnp.float32),
            pltpu.VMEM((_S // 2, _L), jnp.float32),
        ],
    )(logB_flat, pf, pb)
    return alpha, beta, ylin


def _deint(x2):
    return jnp.swapaxes(x2.reshape(_B, _TC, _L), 1, 2)[:, :_S, :]


def _finalize_kernel(alpha_ref, beta_ref, ylin_ref, lp_ref, gamma_ref, xi_ref):
    la = jnp.log(_deint(alpha_ref[...]))
    lb = jnp.log(_deint(beta_ref[...]))

    g = la + lb
    g = g - jnp.max(g, axis=1, keepdims=True)
    g = g - jnp.log(jnp.sum(jnp.exp(g), axis=1, keepdims=True))
    gamma_ref[...] = jnp.swapaxes(g, 1, 2)

    ly = jnp.log(_deint(ylin_ref[...]))
    z = (la[:, :, None, :] + ly[:, None, :, :]
         + lp_ref[...][None, :, :, None]).reshape(_B, _S * _S, _TC)
    z = z - jnp.max(z, axis=1, keepdims=True)
    z = z - jnp.log(jnp.sum(jnp.exp(z), axis=1, keepdims=True))
    xi_ref[...] = jnp.swapaxes(z, 1, 2)


def _run_finalize(alpha, beta, ylin, log_P):
    cs = pl.BlockSpec((_B, _TC * _L), lambda c: (0, c))
    gamma, xi64 = pl.pallas_call(
        _finalize_kernel,
        grid=(_NC,),
        in_specs=[cs, cs, cs, pl.BlockSpec((_S, _S), lambda c: (0, 0))],
        out_specs=[pl.BlockSpec((_B, _TC, _S), lambda c: (0, c, 0)),
                   pl.BlockSpec((_B, _TC, _S * _S), lambda c: (0, c, 0))],
        out_shape=[
            jax.ShapeDtypeStruct((_B, _T, _S), jnp.float32),
            jax.ShapeDtypeStruct((_B, _T - 1, _S * _S), jnp.float32),
        ],
    )(alpha, beta, ylin, log_P)
    return gamma, xi64.reshape(_B, _T - 1, _S, _S)


def kernel(log_B, trans_prob):
    log_B = log_B.astype(jnp.float32)
    log_P = jnp.log(jax.lax.stop_gradient(trans_prob)).astype(jnp.float32)
    alpha, beta, ylin = _run_sc_recur(log_B, trans_prob)
    return _run_finalize(alpha, beta, ylin, log_P)
